# baseline (speedup 1.0000x reference)
import jax
import jax.numpy as jnp
from jax import lax
from jax.experimental import pallas as pl
from jax.experimental.pallas import tpu as pltpu
from jax.experimental.pallas import tpu_sc as plsc

N = 1600000
D = 15
S = 4096
OUT = 5

NC = 2
NS = 16
NW = NC * NS
LANES = 16
ROWS_PER_TILE = N // NW
CHUNK = 400
NCHUNKS = ROWS_PER_TILE // CHUNK


SCROWS = NS * CHUNK
ROWS_PER_SC = N // NC
NSCCHUNKS = ROWS_PER_SC // SCROWS


def _sc_body(x_hbm, batch_hbm, out_hbm, xbufa, xbufb, bbufa, bbufb, acc,
             spxa, spxb, spba, spbb, semx0, semx1, semb0, semb1):
    cid = lax.axis_index("c")
    sid = lax.axis_index("s")
    wid = cid * NS + sid
    iota = lax.iota(jnp.int32, LANES)
    lane_is_feat = iota < D
    ones = jnp.ones((LANES,), jnp.float32)
    zeros = jnp.zeros((LANES,), jnp.float32)
    xbufs, bbufs = [xbufa, xbufb], [bbufa, bbufb]
    spxs, spbs = [spxa, spxb], [spba, spbb]
    semxs, sembs = [semx0, semx1], [semb0, semb1]

    @plsc.parallel_loop(0, LANES * S, step=LANES, unroll=4)
    def _zero(j):
        acc[pl.ds(j, LANES)] = zeros

    def _flush(acc_reg, cur_seg, maskv):
        plsc.addupdate_scatter(
            acc, [jnp.full((LANES,), cur_seg * LANES, jnp.int32) + iota],
            acc_reg, mask=maskv)

    def _hbm_refs(c, slot):
        r0 = cid * ROWS_PER_SC + c * SCROWS
        return (x_hbm.at[pl.ds(r0 * D, SCROWS * D)], spxs[slot], semxs[slot],
                batch_hbm.at[pl.ds(r0, SCROWS)], spbs[slot], sembs[slot])

    def issue(c, slot):
        @pl.when(sid == 0)
        def _():
            xs, xd, xm, bs, bd, bm = _hbm_refs(c, slot)
            pltpu.async_copy(xs, xd, xm)
            pltpu.async_copy(bs, bd, bm)

    def wait(c, slot):
        @pl.when(sid == 0)
        def _():
            xs, xd, xm, bs, bd, bm = _hbm_refs(c, slot)
            pltpu.make_async_copy(xs, xd, xm).wait()
            pltpu.make_async_copy(bs, bd, bm).wait()

    def make_group_body(xbuf, bbuf):
        def group_body(g, carry):
            acc_reg, cur_seg = carry
            g0 = g * LANES
            bvec = bbuf[pl.ds(g0, LANES)]
            in_run = bvec == jnp.full((LANES,), cur_seg)
            all_same = plsc.all_reduce_population_count(in_run)[0] == LANES

            def fast(carry):
                acc_reg, cur_seg = carry
                rows = [xbuf[pl.ds((g0 + k) * D, LANES)] for k in range(LANES)]
                while len(rows) > 1:
                    rows = [a + b for a, b in zip(rows[::2], rows[1::2])]
                s = jnp.where(lane_is_feat, rows[0], jnp.float32(LANES))
                return acc_reg + s, cur_seg

            def slow(carry):
                acc_reg, cur_seg = carry
                for k in range(LANES):
                    row = xbuf[pl.ds((g0 + k) * D, LANES)]
                    vals = jnp.where(lane_is_feat, row, ones)
                    bk = bvec[k]
                    change = bk != cur_seg
                    maskv = jnp.full((LANES,), change)
                    _flush(acc_reg, cur_seg, maskv)
                    acc_reg = jnp.where(maskv, 0.0, acc_reg)
                    cur_seg = jnp.where(change, bk, cur_seg)
                    acc_reg = acc_reg + vals
                return acc_reg, cur_seg

            return lax.cond(all_same, fast, slow, (acc_reg, cur_seg))
        return group_body

    def process(c, slot, issue_next):
        wait(c, slot)
        plsc.subcore_barrier()
        pltpu.sync_copy(spxs[slot].at[pl.ds(sid * CHUNK * D, CHUNK * D)],
                        xbufs[slot].at[pl.ds(0, CHUNK * D)])
        pltpu.sync_copy(spbs[slot].at[pl.ds(sid * CHUNK, CHUNK)], bbufs[slot])
        plsc.subcore_barrier()
        if issue_next:
            issue(c + 1, 1 - slot)
        cur_seg0 = bbufs[slot][pl.ds(0, LANES)][0]
        acc_reg, cur_seg = lax.fori_loop(
            0, CHUNK // LANES, make_group_body(xbufs[slot], bbufs[slot]),
            (zeros, cur_seg0))
        _flush(acc_reg, cur_seg, None)

    issue(0, 0)

    def pair_body(t, _):
        process(2 * t, 0, True)
        process(2 * t + 1, 1, True)
        return 0
    lax.fori_loop(0, NSCCHUNKS // 2, pair_body, 0)
    process(NSCCHUNKS - 1, (NSCCHUNKS - 1) % 2, False)

    pltpu.sync_copy(acc, out_hbm.at[wid])


_sc_segment_sum = pl.kernel(
    _sc_body,
    out_type=jax.ShapeDtypeStruct((NW, S * LANES), jnp.float32),
    mesh=plsc.VectorSubcoreMesh(core_axis_name="c", subcore_axis_name="s"),
    compiler_params=pltpu.CompilerParams(needs_layout_passes=False),
    scratch_types=[
        pltpu.VMEM((CHUNK * D + LANES,), jnp.float32),
        pltpu.VMEM((CHUNK * D + LANES,), jnp.float32),
        pltpu.VMEM((CHUNK,), jnp.int32),
        pltpu.VMEM((CHUNK,), jnp.int32),
        pltpu.VMEM((LANES * S,), jnp.float32),
        pltpu.VMEM_SHARED((SCROWS * D,), jnp.float32),
        pltpu.VMEM_SHARED((SCROWS * D,), jnp.float32),
        pltpu.VMEM_SHARED((SCROWS,), jnp.int32),
        pltpu.VMEM_SHARED((SCROWS,), jnp.int32),
        pltpu.SemaphoreType.DMA,
        pltpu.SemaphoreType.DMA,
        pltpu.SemaphoreType.DMA,
        pltpu.SemaphoreType.DMA,
    ],
)


def _tc_tail_body(p_ref, w_ref, b_ref, o_ref, acc_ref):
    i = pl.program_id(0)

    @pl.when(i == 0)
    def _init():
        acc_ref[...] = p_ref[0]

    @pl.when(i > 0)
    def _accum():
        acc_ref[...] += p_ref[0]

    @pl.when(i == NW - 1)
    def _finish():
        s = acc_ref[...]
        counts = s[:, D]
        mean = s[:, :D] / jnp.maximum(counts, 1.0)[:, None]
        out = lax.dot_general(mean, w_ref[...], (((1,), (1,)), ((), ())),
                              preferred_element_type=jnp.float32)
        o_ref[...] = jnp.where(counts[:, None] > 0, out + b_ref[...][None, :], 0.0)


_tc_tail = pl.pallas_call(
    _tc_tail_body,
    grid=(NW,),
    in_specs=[
        pl.BlockSpec((1, S, LANES), lambda i: (i, 0, 0)),
        pl.BlockSpec((OUT, D), lambda i: (0, 0)),
        pl.BlockSpec((OUT,), lambda i: (0,)),
    ],
    out_specs=pl.BlockSpec((S, OUT), lambda i: (0, 0)),
    scratch_shapes=[pltpu.VMEM((S, LANES), jnp.float32)],
    out_shape=jax.ShapeDtypeStruct((S, OUT), jnp.float32),
)


def kernel(x, batch, W, b):
    partials = _sc_segment_sum(x.reshape(N * D), batch.astype(jnp.int32))
    return _tc_tail(partials.reshape(NW, S, LANES), W, b)


# speedup vs baseline: 1.0701x; 1.0701x over previous
import jax
import jax.numpy as jnp
from jax import lax
from jax.experimental import pallas as pl
from jax.experimental.pallas import tpu as pltpu
from jax.experimental.pallas import tpu_sc as plsc

N = 1600000
D = 15
S = 4096
OUT = 5
YD = 8

NC = 2
NS = 16
NW = NC * NS
LANES = 16
ROWS_PER_TILE = N // NW
CHUNK = 2000
NCHUNKS = ROWS_PER_TILE // CHUNK
NPAIRS = NCHUNKS // 2

PROJ_PACK = 8
PROJ_BLK = 1000


def _tc_proj_body(x_ref, wbd_ref, o_ref):
    y = jnp.dot(x_ref[...], wbd_ref[...], preferred_element_type=jnp.float32)
    col = lax.broadcasted_iota(jnp.int32, y.shape, 1)
    o_ref[...] = jnp.where(col % YD == OUT, 1.0, y)


_tc_project = pl.pallas_call(
    _tc_proj_body,
    grid=(N // PROJ_PACK // PROJ_BLK,),
    in_specs=[
        pl.BlockSpec((PROJ_BLK, PROJ_PACK * D), lambda i: (i, 0)),
        pl.BlockSpec((PROJ_PACK * D, PROJ_PACK * YD), lambda i: (0, 0)),
    ],
    out_specs=pl.BlockSpec((PROJ_BLK, PROJ_PACK * YD), lambda i: (i, 0)),
    out_shape=jax.ShapeDtypeStruct((N // PROJ_PACK, PROJ_PACK * YD),
                                   jnp.float32),
)


def _sc_body(y_hbm, batch_hbm, out_hbm, ybufa, ybufb, bbufa, bbufb, acc,
             semy0, semy1, semb0, semb1):
    wid = lax.axis_index("s") * NC + lax.axis_index("c")
    base_row = wid * ROWS_PER_TILE
    iota = lax.iota(jnp.int32, LANES)
    iota7 = jnp.bitwise_and(iota, 7)
    low_half = iota < YD
    zeros = jnp.zeros((LANES,), jnp.float32)
    ybufs, bbufs = [ybufa, ybufb], [bbufa, bbufb]
    semys, sembs = [semy0, semy1], [semb0, semb1]

    @plsc.parallel_loop(0, S * YD, step=LANES, unroll=4)
    def _zero(j):
        acc[pl.ds(j, LANES)] = zeros

    def _refs(c, slot):
        r0 = base_row + c * CHUNK
        return (y_hbm.at[pl.ds(r0 * YD, CHUNK * YD)], ybufs[slot], semys[slot],
                batch_hbm.at[pl.ds(r0, CHUNK)], bbufs[slot], sembs[slot])

    def issue(c, slot):
        ys, yd_, ym, bs, bd, bm = _refs(c, slot)
        pltpu.async_copy(ys, yd_, ym)
        pltpu.async_copy(bs, bd, bm)

    def wait(c, slot):
        ys, yd_, ym, bs, bd, bm = _refs(c, slot)
        pltpu.make_async_copy(ys, yd_, ym).wait()
        pltpu.make_async_copy(bs, bd, bm).wait()

    def _flush(acc_reg, cur_seg):
        idx = jnp.full((LANES,), cur_seg * YD, jnp.int32) + iota7
        plsc.addupdate_scatter(acc, [idx], acc_reg)

    def compute(c, slot):
        ybuf, bbuf = ybufs[slot], bbufs[slot]

        def group_body(g, carry):
            acc_reg, cur_seg = carry
            g0 = g * LANES
            bvec = bbuf[pl.ds(g0, LANES)]
            in_run = bvec == jnp.full((LANES,), cur_seg)
            all_same = plsc.all_reduce_population_count(in_run)[0] == LANES

            def fast(carry):
                acc_reg, cur_seg = carry
                vs = [ybuf[pl.ds(g0 * YD + LANES * k, LANES)]
                      for k in range(8)]
                while len(vs) > 1:
                    vs = [a + b for a, b in zip(vs[::2], vs[1::2])]
                return acc_reg + vs[0], cur_seg

            def slow(carry):
                acc_reg, cur_seg = carry
                _flush(acc_reg, cur_seg)
                for k in range(8):
                    v = ybuf[pl.ds(g0 * YD + LANES * k, LANES)]
                    sa = bvec[2 * k]
                    sb = bvec[2 * k + 1]
                    segv = jnp.where(low_half, jnp.full((LANES,), sa),
                                     jnp.full((LANES,), sb))
                    plsc.addupdate_scatter(acc, [segv * YD + iota7], v)
                return zeros, bvec[LANES - 1]

            return lax.cond(all_same, fast, slow, (acc_reg, cur_seg))

        cur_seg0 = bbuf[pl.ds(0, LANES)][0]
        acc_reg, cur_seg = lax.fori_loop(
            0, CHUNK // LANES, group_body, (zeros, cur_seg0))
        _flush(acc_reg, cur_seg)

    issue(0, 0)

    def pair_body(t, _):
        issue(2 * t + 1, 1)
        wait(2 * t, 0)
        compute(2 * t, 0)
        issue(2 * t + 2, 0)
        wait(2 * t + 1, 1)
        compute(2 * t + 1, 1)
        return 0
    lax.fori_loop(0, NPAIRS, pair_body, 0)
    wait(NCHUNKS - 1, 0)
    compute(NCHUNKS - 1, 0)

    pltpu.sync_copy(acc, out_hbm.at[wid])


_sc_segment_sum = pl.kernel(
    _sc_body,
    out_type=jax.ShapeDtypeStruct((NW, S * YD), jnp.float32),
    mesh=plsc.VectorSubcoreMesh(core_axis_name="c", subcore_axis_name="s"),
    compiler_params=pltpu.CompilerParams(needs_layout_passes=False),
    scratch_types=[
        pltpu.VMEM((CHUNK * YD,), jnp.float32),
        pltpu.VMEM((CHUNK * YD,), jnp.float32),
        pltpu.VMEM((CHUNK,), jnp.int32),
        pltpu.VMEM((CHUNK,), jnp.int32),
        pltpu.VMEM((S * YD,), jnp.float32),
        pltpu.SemaphoreType.DMA,
        pltpu.SemaphoreType.DMA,
        pltpu.SemaphoreType.DMA,
        pltpu.SemaphoreType.DMA,
    ],
)


def _tc_tail_body(p_ref, b_ref, o_ref, acc_ref):
    i = pl.program_id(0)

    @pl.when(i == 0)
    def _init():
        acc_ref[...] = p_ref[0]

    @pl.when(i > 0)
    def _accum():
        acc_ref[...] += p_ref[0]

    @pl.when(i == NW - 1)
    def _finish():
        s = acc_ref[...]
        counts = s[:, OUT]
        mean = s[:, :OUT] / jnp.maximum(counts, 1.0)[:, None]
        o_ref[...] = jnp.where(counts[:, None] > 0,
                               mean + b_ref[...][None, :], 0.0)


_tc_tail = pl.pallas_call(
    _tc_tail_body,
    grid=(NW,),
    in_specs=[
        pl.BlockSpec((1, S, YD), lambda i: (i, 0, 0)),
        pl.BlockSpec((OUT,), lambda i: (0,)),
    ],
    out_specs=pl.BlockSpec((S, OUT), lambda i: (0, 0)),
    scratch_shapes=[pltpu.VMEM((S, YD), jnp.float32)],
    out_shape=jax.ShapeDtypeStruct((S, OUT), jnp.float32),
)


def kernel(x, batch, W, b):
    wt8 = jnp.pad(W.T, ((0, 0), (0, YD - OUT)))
    wbd = jnp.kron(jnp.eye(PROJ_PACK, dtype=x.dtype), wt8)
    y = _tc_project(x.reshape(N // PROJ_PACK, PROJ_PACK * D), wbd)
    partials = _sc_segment_sum(y.reshape(N * YD), batch.astype(jnp.int32))
    return _tc_tail(partials.reshape(NW, S, YD), b)
